# baseline (speedup 1.0000x reference)
import functools

import jax
import jax.numpy as jnp
from jax import lax
from jax.experimental import pallas as pl
from jax.experimental.pallas import tpu as pltpu
from jax.experimental.pallas import tpu_sc as plsc

NC, NS, L = 2, 16, 16
NW = NC * NS
CHUNK = 16


def _sc_interp(table, pc, *, B, DHW, C, n_pad):
    per_tile = n_pad // NW
    n_chunks = per_tile // CHUNK
    mesh = plsc.VectorSubcoreMesh(core_axis_name="c", subcore_axis_name="s",
                                  num_cores=NC, num_subcores=NS)

    @functools.partial(
        pl.kernel,
        out_type=jax.ShapeDtypeStruct((B * n_pad, C), jnp.float32),
        mesh=mesh,
        scratch_types=[
            [pltpu.VMEM((per_tile,), jnp.float32) for _ in range(3)],
            pltpu.VMEM((2, 8 * CHUNK), jnp.int32),
            pltpu.VMEM((2, 8, CHUNK + L), jnp.float32),
            pltpu.VMEM((2, 8 * CHUNK, C), jnp.float32),
            pltpu.VMEM((2, CHUNK, C), jnp.float32),
            pltpu.SemaphoreType.DMA,
            pltpu.SemaphoreType.DMA,
            pltpu.SemaphoreType.DMA,
            pltpu.SemaphoreType.DMA,
        ],
    )
    def k(table_hbm, pc_hbm, out_hbm, coords_v, idx_v, w_v, rows_v, out_v,
          sem_a, sem_b, osem_a, osem_b):
        wid = lax.axis_index("s") * NC + lax.axis_index("c")
        tile_base = wid * per_tile
        sems = (sem_a, sem_b)
        osems = (osem_a, osem_b)

        for b in range(B):
            for i in range(3):
                pltpu.sync_copy(
                    pc_hbm.at[pl.ds((b * 3 + i) * n_pad + tile_base, per_tile)],
                    coords_v[i])

            def fill_idx(g, par, b=b):
                s = pl.ds(g * CHUNK, L)
                pz = coords_v[0][s]
                py = coords_v[1][s]
                px = coords_v[2][s]

                def split(f):
                    f = jnp.clip(f * 47.0, 0.0, 47.0)
                    i0 = f.astype(jnp.int32)
                    w1 = f - i0.astype(jnp.float32)
                    i1 = jnp.minimum(i0 + 1, 47)
                    return i0, i1, 1.0 - w1, w1

                z0, z1, wz0, wz1 = split(pz)
                y0, y1, wy0, wy1 = split(py)
                x0, x1, wx0, wx1 = split(px)
                zy = [(z0 * 48 + y0, wz0 * wy0), (z0 * 48 + y1, wz0 * wy1),
                      (z1 * 48 + y0, wz1 * wy0), (z1 * 48 + y1, wz1 * wy1)]
                sl = pl.ds(0, L)
                ci = 0
                for lzy, wzy in zy:
                    for lx, wx in ((x0, wx0), (x1, wx1)):
                        idx_v[par, pl.ds(ci * CHUNK, L)] = (
                            lzy * 48 + lx + (b * DHW))
                        w_v[par, ci, sl] = wzy * wx
                        ci += 1

            def issue(par):
                pltpu.async_copy(table_hbm.at[idx_v.at[par]],
                                 rows_v.at[par], sems[par])

            def drain(par):
                pltpu.make_async_copy(table_hbm.at[idx_v.at[par]],
                                      rows_v.at[par], sems[par]).wait()

            def out_slice(g, b=b):
                return out_hbm.at[pl.ds(b * n_pad + tile_base + g * CHUNK,
                                        CHUNK)]

            def blend_store(g, par, b=b):
                @pl.when(g >= 2)
                def _():
                    pltpu.make_async_copy(out_v.at[par], out_slice(g - 2),
                                          osems[par]).wait()

                def blend(p, carry2):
                    ws = [w_v[par, ci, pl.ds(p, L)][0] for ci in range(8)]
                    for v in range(C // L):
                        t = [rows_v[par, ci * CHUNK + p, pl.ds(v * L, L)]
                             * ws[ci] for ci in range(8)]
                        out_v[par, p, pl.ds(v * L, L)] = (
                            (t[0] + t[1]) + (t[2] + t[3])) + (
                            (t[4] + t[5]) + (t[6] + t[7]))
                    return carry2

                lax.fori_loop(0, CHUNK, blend, 0, unroll=2)
                pltpu.async_copy(out_v.at[par], out_slice(g), osems[par])

            fill_idx(0, 0)
            issue(0)

            def chunk_body(g0, carry):
                for par in range(2):
                    g = g0 * 2 + par

                    @pl.when(g + 1 < n_chunks)
                    def _():
                        fill_idx(g + 1, 1 - par)
                        issue(1 - par)

                    drain(par)
                    blend_store(g, par)
                return carry

            lax.fori_loop(0, n_chunks // 2, chunk_body, 0, unroll=False)

            pltpu.make_async_copy(out_v.at[0], out_slice(n_chunks - 2),
                                  osems[0]).wait()
            pltpu.make_async_copy(out_v.at[1], out_slice(n_chunks - 1),
                                  osems[1]).wait()

    return k(table, pc)


def kernel(voxel_features, voxel_coords, point_coords):
    del voxel_coords
    B, C, D, H, W = voxel_features.shape
    N = point_coords.shape[1]
    DHW = D * H * W
    per_tile = -(-N // NW)
    per_tile = -(-per_tile // (2 * CHUNK)) * (2 * CHUNK)
    n_pad = per_tile * NW

    table = voxel_features.transpose(0, 2, 3, 4, 1).reshape(B * DHW, C)
    pc = point_coords.transpose(0, 2, 1)
    pc = jnp.pad(pc, ((0, 0), (0, 0), (0, n_pad - N))).reshape(B * 3 * n_pad)
    out = _sc_interp(table, pc, B=B, DHW=DHW, C=C, n_pad=n_pad)
    return out.reshape(B, n_pad, C)[:, :N, :]


# speedup vs baseline: 1.0607x; 1.0607x over previous
import functools

import jax
import jax.numpy as jnp
from jax import lax
from jax.experimental import pallas as pl
from jax.experimental.pallas import tpu as pltpu
from jax.experimental.pallas import tpu_sc as plsc

NC, NS, L = 2, 16, 16
NW = NC * NS
CHUNK = 16


def _sc_interp(table, pc, *, B, DHW, C, n_pad):
    per_tile = n_pad // NW
    n_chunks = per_tile // CHUNK
    mesh = plsc.VectorSubcoreMesh(core_axis_name="c", subcore_axis_name="s",
                                  num_cores=NC, num_subcores=NS)

    @functools.partial(
        pl.kernel,
        out_type=jax.ShapeDtypeStruct((B * n_pad, C), jnp.float32),
        mesh=mesh,
        scratch_types=[
            [pltpu.VMEM((per_tile,), jnp.float32) for _ in range(3)],
            pltpu.VMEM((3, 8, CHUNK), jnp.int32),
            pltpu.VMEM((3, 8, CHUNK + L), jnp.float32),
            pltpu.VMEM((3, 8, CHUNK, C), jnp.float32),
            pltpu.VMEM((3, CHUNK, C), jnp.float32),
            pltpu.SemaphoreType.DMA,
            pltpu.SemaphoreType.DMA,
            pltpu.SemaphoreType.DMA,
            pltpu.SemaphoreType.DMA,
            pltpu.SemaphoreType.DMA,
            pltpu.SemaphoreType.DMA,
        ],
    )
    def k(table_hbm, pc_hbm, out_hbm, coords_v, idx_v, w_v, rows_v, out_v,
          sem_a, sem_b, sem_c, osem_a, osem_b, osem_c):
        wid = lax.axis_index("s") * NC + lax.axis_index("c")
        tile_base = wid * per_tile
        sems = (sem_a, sem_b, sem_c)
        osems = (osem_a, osem_b, osem_c)

        for b in range(B):
            for i in range(3):
                pltpu.sync_copy(
                    pc_hbm.at[pl.ds((b * 3 + i) * n_pad + tile_base, per_tile)],
                    coords_v[i])

            def fill_idx(g, par, b=b):
                s = pl.ds(g * CHUNK, L)
                pz = coords_v[0][s]
                py = coords_v[1][s]
                px = coords_v[2][s]

                def split(f):
                    f = jnp.clip(f * 47.0, 0.0, 47.0)
                    i0 = f.astype(jnp.int32)
                    w1 = f - i0.astype(jnp.float32)
                    i1 = jnp.minimum(i0 + 1, 47)
                    return i0, i1, 1.0 - w1, w1

                z0, z1, wz0, wz1 = split(pz)
                y0, y1, wy0, wy1 = split(py)
                x0, x1, wx0, wx1 = split(px)
                zy = [(z0 * 48 + y0, wz0 * wy0), (z0 * 48 + y1, wz0 * wy1),
                      (z1 * 48 + y0, wz1 * wy0), (z1 * 48 + y1, wz1 * wy1)]
                sl = pl.ds(0, L)
                ci = 0
                for lzy, wzy in zy:
                    for lx, wx in ((x0, wx0), (x1, wx1)):
                        idx_v[par, ci, sl] = lzy * 48 + lx + (b * DHW)
                        w_v[par, ci, sl] = wzy * wx
                        ci += 1

            def issue(par):
                for ci in range(8):
                    pltpu.async_copy(table_hbm.at[idx_v.at[par, ci]],
                                     rows_v.at[par, ci], sems[par])

            def drain(par):
                for ci in range(8):
                    pltpu.make_async_copy(table_hbm.at[idx_v.at[par, ci]],
                                          rows_v.at[par, ci],
                                          sems[par]).wait()

            def out_slice(g, b=b):
                return out_hbm.at[pl.ds(b * n_pad + tile_base + g * CHUNK,
                                        CHUNK)]

            def blend_store(g, par, opar, b=b):
                @pl.when(g >= 3)
                def _():
                    pltpu.make_async_copy(out_v.at[opar], out_slice(g - 3),
                                          osems[opar]).wait()

                def blend(p, carry2):
                    ws = [w_v[par, ci, pl.ds(p, L)][0] for ci in range(8)]
                    for v in range(C // L):
                        t = [rows_v[par, ci, p, pl.ds(v * L, L)]
                             * ws[ci] for ci in range(8)]
                        out_v[opar, p, pl.ds(v * L, L)] = (
                            (t[0] + t[1]) + (t[2] + t[3])) + (
                            (t[4] + t[5]) + (t[6] + t[7]))
                    return carry2

                lax.fori_loop(0, CHUNK, blend, 0, unroll=2)
                pltpu.async_copy(out_v.at[opar], out_slice(g), osems[opar])

            fill_idx(0, 0)
            issue(0)
            fill_idx(1, 1)
            issue(1)

            def chunk_body(g0, carry):
                for par in range(3):
                    g = g0 * 3 + par

                    @pl.when(g + 2 < n_chunks)
                    def _():
                        fill_idx(g + 2, (par + 2) % 3)
                        issue((par + 2) % 3)

                    drain(par)
                    blend_store(g, par, par)
                return carry

            lax.fori_loop(0, n_chunks // 3, chunk_body, 0, unroll=False)

            for j in range(3):
                pltpu.make_async_copy(out_v.at[j],
                                      out_slice(n_chunks - 3 + j),
                                      osems[j]).wait()

    return k(table, pc)


def kernel(voxel_features, voxel_coords, point_coords):
    del voxel_coords
    B, C, D, H, W = voxel_features.shape
    N = point_coords.shape[1]
    DHW = D * H * W
    per_tile = -(-N // NW)
    per_tile = -(-per_tile // (3 * CHUNK)) * (3 * CHUNK)
    n_pad = per_tile * NW

    table = voxel_features.transpose(0, 2, 3, 4, 1).reshape(B * DHW, C)
    pc = point_coords.transpose(0, 2, 1)
    pc = jnp.pad(pc, ((0, 0), (0, 0), (0, n_pad - N))).reshape(B * 3 * n_pad)
    out = _sc_interp(table, pc, B=B, DHW=DHW, C=C, n_pad=n_pad)
    return out.reshape(B, n_pad, C)[:, :N, :]


# speedup vs baseline: 1.0873x; 1.0250x over previous
import functools

import jax
import jax.numpy as jnp
from jax import lax
from jax.experimental import pallas as pl
from jax.experimental.pallas import tpu as pltpu
from jax.experimental.pallas import tpu_sc as plsc

NC, NS, L = 2, 16, 16
NW = NC * NS
CHUNK = 16


def _sc_interp(table, pc, *, B, DHW, C, n_pad):
    per_tile = n_pad // NW
    n_chunks = per_tile // CHUNK
    mesh = plsc.VectorSubcoreMesh(core_axis_name="c", subcore_axis_name="s",
                                  num_cores=NC, num_subcores=NS)

    @functools.partial(
        pl.kernel,
        out_type=jax.ShapeDtypeStruct((B * n_pad, C), jnp.float32),
        mesh=mesh,
        scratch_types=[
            [pltpu.VMEM((per_tile,), jnp.float32) for _ in range(3)],
            pltpu.VMEM((2, 8, CHUNK), jnp.int32),
            pltpu.VMEM((2, 8, CHUNK + L), jnp.float32),
            pltpu.VMEM((2, 8, CHUNK, C), jnp.float32),
            pltpu.VMEM((2, CHUNK, C), jnp.float32),
            pltpu.SemaphoreType.DMA,
            pltpu.SemaphoreType.DMA,
            pltpu.SemaphoreType.DMA,
            pltpu.SemaphoreType.DMA,
        ],
    )
    def k(table_hbm, pc_hbm, out_hbm, coords_v, idx_v, w_v, rows_v, out_v,
          sem_a, sem_b, osem_a, osem_b):
        wid = lax.axis_index("s") * NC + lax.axis_index("c")
        tile_base = wid * per_tile
        sems = (sem_a, sem_b)
        osems = (osem_a, osem_b)

        for b in range(B):
            for i in range(3):
                pltpu.sync_copy(
                    pc_hbm.at[pl.ds((b * 3 + i) * n_pad + tile_base, per_tile)],
                    coords_v[i])

            def fill_idx(g, par, b=b):
                s = pl.ds(g * CHUNK, L)
                pz = coords_v[0][s]
                py = coords_v[1][s]
                px = coords_v[2][s]

                def split(f):
                    f = jnp.clip(f * 47.0, 0.0, 47.0)
                    i0 = f.astype(jnp.int32)
                    w1 = f - i0.astype(jnp.float32)
                    i1 = jnp.minimum(i0 + 1, 47)
                    return i0, i1, 1.0 - w1, w1

                z0, z1, wz0, wz1 = split(pz)
                y0, y1, wy0, wy1 = split(py)
                x0, x1, wx0, wx1 = split(px)
                zy = [(z0 * 48 + y0, wz0 * wy0), (z0 * 48 + y1, wz0 * wy1),
                      (z1 * 48 + y0, wz1 * wy0), (z1 * 48 + y1, wz1 * wy1)]
                sl = pl.ds(0, L)
                ci = 0
                for lzy, wzy in zy:
                    for lx, wx in ((x0, wx0), (x1, wx1)):
                        idx_v[par, ci, sl] = lzy * 48 + lx + (b * DHW)
                        w_v[par, ci, sl] = wzy * wx
                        ci += 1

            H = CHUNK // 2

            def issue(par):
                for ci in range(8):
                    for h in range(2):
                        pltpu.async_copy(
                            table_hbm.at[idx_v.at[par, ci, pl.ds(h * H, H)]],
                            rows_v.at[par, ci, pl.ds(h * H, H)], sems[par])

            def drain(par):
                for ci in range(8):
                    for h in range(2):
                        pltpu.make_async_copy(
                            table_hbm.at[idx_v.at[par, ci, pl.ds(h * H, H)]],
                            rows_v.at[par, ci, pl.ds(h * H, H)],
                            sems[par]).wait()

            def out_slice(g, b=b):
                return out_hbm.at[pl.ds(b * n_pad + tile_base + g * CHUNK,
                                        CHUNK)]

            def blend_store(g, par, opar, b=b):
                @pl.when(g >= 2)
                def _():
                    pltpu.make_async_copy(out_v.at[opar], out_slice(g - 2),
                                          osems[opar]).wait()

                def blend(p, carry2):
                    ws = [w_v[par, ci, pl.ds(p, L)][0] for ci in range(8)]
                    for v in range(C // L):
                        t = [rows_v[par, ci, p, pl.ds(v * L, L)]
                             * ws[ci] for ci in range(8)]
                        out_v[opar, p, pl.ds(v * L, L)] = (
                            (t[0] + t[1]) + (t[2] + t[3])) + (
                            (t[4] + t[5]) + (t[6] + t[7]))
                    return carry2

                lax.fori_loop(0, CHUNK, blend, 0, unroll=2)
                pltpu.async_copy(out_v.at[opar], out_slice(g), osems[opar])

            fill_idx(0, 0)
            issue(0)

            def chunk_body(g0, carry):
                for par in range(2):
                    g = g0 * 2 + par

                    @pl.when(g + 1 < n_chunks)
                    def _():
                        fill_idx(g + 1, 1 - par)
                        issue(1 - par)

                    drain(par)
                    blend_store(g, par, par)
                return carry

            lax.fori_loop(0, n_chunks // 2, chunk_body, 0, unroll=False)

            pltpu.make_async_copy(out_v.at[0], out_slice(n_chunks - 2),
                                  osems[0]).wait()
            pltpu.make_async_copy(out_v.at[1], out_slice(n_chunks - 1),
                                  osems[1]).wait()

    return k(table, pc)


def kernel(voxel_features, voxel_coords, point_coords):
    del voxel_coords
    B, C, D, H, W = voxel_features.shape
    N = point_coords.shape[1]
    DHW = D * H * W
    per_tile = -(-N // NW)
    per_tile = -(-per_tile // (2 * CHUNK)) * (2 * CHUNK)
    n_pad = per_tile * NW

    table = voxel_features.transpose(0, 2, 3, 4, 1).reshape(B * DHW, C)
    pc = point_coords.transpose(0, 2, 1)
    pc = jnp.pad(pc, ((0, 0), (0, 0), (0, n_pad - N))).reshape(B * 3 * n_pad)
    out = _sc_interp(table, pc, B=B, DHW=DHW, C=C, n_pad=n_pad)
    return out.reshape(B, n_pad, C)[:, :N, :]


# speedup vs baseline: 1.1162x; 1.0266x over previous
import functools

import jax
import jax.numpy as jnp
from jax import lax
from jax.experimental import pallas as pl
from jax.experimental.pallas import tpu as pltpu
from jax.experimental.pallas import tpu_sc as plsc

NC, NS, L = 2, 16, 16
NW = NC * NS
CHUNK = 16


def _sc_interp(table, pc, *, B, DHW, C, n_pad):
    per_tile = n_pad // NW
    n_chunks = per_tile // CHUNK
    mesh = plsc.VectorSubcoreMesh(core_axis_name="c", subcore_axis_name="s",
                                  num_cores=NC, num_subcores=NS)

    @functools.partial(
        pl.kernel,
        out_type=jax.ShapeDtypeStruct((B * n_pad, C), jnp.float32),
        mesh=mesh,
        scratch_types=[
            [pltpu.VMEM((per_tile,), jnp.float32) for _ in range(3)],
            pltpu.VMEM((2, 8, CHUNK), jnp.int32),
            pltpu.VMEM((2, 8, CHUNK + L), jnp.float32),
            pltpu.VMEM((2, 8, CHUNK, C), jnp.float32),
            pltpu.VMEM((2, CHUNK, C), jnp.float32),
            pltpu.SemaphoreType.DMA,
            pltpu.SemaphoreType.DMA,
            pltpu.SemaphoreType.DMA,
            pltpu.SemaphoreType.DMA,
        ],
    )
    def k(table_hbm, pc_hbm, out_hbm, coords_v, idx_v, w_v, rows_v, out_v,
          sem_a, sem_b, osem_a, osem_b):
        wid = lax.axis_index("s") * NC + lax.axis_index("c")
        tile_base = wid * per_tile
        sems = (sem_a, sem_b)
        osems = (osem_a, osem_b)

        for b in range(B):
            for i in range(3):
                pltpu.sync_copy(
                    pc_hbm.at[pl.ds((b * 3 + i) * n_pad + tile_base, per_tile)],
                    coords_v[i])

            def fill_idx(g, par, b=b):
                s = pl.ds(g * CHUNK, L)
                pz = coords_v[0][s]
                py = coords_v[1][s]
                px = coords_v[2][s]

                def split(f):
                    f = jnp.clip(f * 47.0, 0.0, 47.0)
                    i0 = f.astype(jnp.int32)
                    w1 = f - i0.astype(jnp.float32)
                    i1 = jnp.minimum(i0 + 1, 47)
                    return i0, i1, 1.0 - w1, w1

                z0, z1, wz0, wz1 = split(pz)
                y0, y1, wy0, wy1 = split(py)
                x0, x1, wx0, wx1 = split(px)
                zy = [(z0 * 48 + y0, wz0 * wy0), (z0 * 48 + y1, wz0 * wy1),
                      (z1 * 48 + y0, wz1 * wy0), (z1 * 48 + y1, wz1 * wy1)]
                sl = pl.ds(0, L)
                ci = 0
                for lzy, wzy in zy:
                    for lx, wx in ((x0, wx0), (x1, wx1)):
                        idx_v[par, ci, sl] = lzy * 48 + lx + (b * DHW)
                        w_v[par, ci, sl] = wzy * wx
                        ci += 1

            def issue(par):
                for ci in range(8):
                    pltpu.async_copy(table_hbm.at[idx_v.at[par, ci]],
                                     rows_v.at[par, ci], sems[par])

            def drain(par):
                for ci in range(8):
                    pltpu.make_async_copy(table_hbm.at[idx_v.at[par, ci]],
                                          rows_v.at[par, ci],
                                          sems[par]).wait()

            def out_slice(g, b=b):
                return out_hbm.at[pl.ds(b * n_pad + tile_base + g * CHUNK,
                                        CHUNK)]

            def blend_store(g, par, opar, b=b):
                @pl.when(g >= 2)
                def _():
                    pltpu.make_async_copy(out_v.at[opar], out_slice(g - 2),
                                          osems[opar]).wait()

                def blend(p, carry2):
                    ws = [w_v[par, ci, pl.ds(p, L)][0] for ci in range(8)]
                    for v in range(C // L):
                        t = [rows_v[par, ci, p, pl.ds(v * L, L)]
                             * ws[ci] for ci in range(8)]
                        out_v[opar, p, pl.ds(v * L, L)] = (
                            (t[0] + t[1]) + (t[2] + t[3])) + (
                            (t[4] + t[5]) + (t[6] + t[7]))
                    return carry2

                lax.fori_loop(0, CHUNK, blend, 0, unroll=2)
                pltpu.async_copy(out_v.at[opar], out_slice(g), osems[opar])

            fill_idx(0, 0)
            issue(0)

            def chunk_body(g0, carry):
                for par in range(2):
                    g = g0 * 2 + par

                    @pl.when(g + 1 < n_chunks)
                    def _():
                        fill_idx(g + 1, 1 - par)
                        issue(1 - par)

                    drain(par)
                    blend_store(g, par, par)
                return carry

            lax.fori_loop(0, n_chunks // 2, chunk_body, 0, unroll=False)

            pltpu.make_async_copy(out_v.at[0], out_slice(n_chunks - 2),
                                  osems[0]).wait()
            pltpu.make_async_copy(out_v.at[1], out_slice(n_chunks - 1),
                                  osems[1]).wait()

    return k(table, pc)


def kernel(voxel_features, voxel_coords, point_coords):
    del voxel_coords
    B, C, D, H, W = voxel_features.shape
    N = point_coords.shape[1]
    DHW = D * H * W
    per_tile = -(-N // NW)
    per_tile = -(-per_tile // (2 * CHUNK)) * (2 * CHUNK)
    n_pad = per_tile * NW

    table = voxel_features.transpose(0, 2, 3, 4, 1).reshape(B * DHW, C)
    pc = point_coords.transpose(0, 2, 1)
    pc = jnp.pad(pc, ((0, 0), (0, 0), (0, n_pad - N))).reshape(B * 3 * n_pad)
    out = _sc_interp(table, pc, B=B, DHW=DHW, C=C, n_pad=n_pad)
    return out.reshape(B, n_pad, C)[:, :N, :]


# speedup vs baseline: 1.1205x; 1.0039x over previous
import functools

import jax
import jax.numpy as jnp
from jax import lax
from jax.experimental import pallas as pl
from jax.experimental.pallas import tpu as pltpu
from jax.experimental.pallas import tpu_sc as plsc

NC, NS, L = 2, 16, 16
NW = NC * NS
CHUNK = 16


def _sc_interp(table, pc, *, B, D, H, W, C, n_pad):
    DHW = D * H * W
    per_tile = n_pad // NW
    n_chunks = per_tile // CHUNK
    mesh = plsc.VectorSubcoreMesh(core_axis_name="c", subcore_axis_name="s",
                                  num_cores=NC, num_subcores=NS)

    @functools.partial(
        pl.kernel,
        out_type=jax.ShapeDtypeStruct((B * n_pad, C), jnp.float32),
        mesh=mesh,
        scratch_types=[
            [pltpu.VMEM((per_tile,), jnp.float32) for _ in range(3)],
            pltpu.VMEM((2, 8, CHUNK), jnp.int32),
            pltpu.VMEM((2, 8, CHUNK + L), jnp.float32),
            pltpu.VMEM((2, 8, CHUNK, C), jnp.float32),
            pltpu.VMEM((2, CHUNK, C), jnp.float32),
            pltpu.SemaphoreType.DMA,
            pltpu.SemaphoreType.DMA,
            pltpu.SemaphoreType.DMA,
            pltpu.SemaphoreType.DMA,
        ],
    )
    def k(table_hbm, pc_hbm, out_hbm, coords_v, idx_v, w_v, rows_v, out_v,
          sem_a, sem_b, osem_a, osem_b):
        wid = lax.axis_index("s") * NC + lax.axis_index("c")
        tile_base = wid * per_tile
        sems = (sem_a, sem_b)
        osems = (osem_a, osem_b)

        for b in range(B):
            for i in range(3):
                pltpu.sync_copy(
                    pc_hbm.at[pl.ds((b * 3 + i) * n_pad + tile_base, per_tile)],
                    coords_v[i])

            def fill_idx(g, par, b=b):
                s = pl.ds(g * CHUNK, L)
                pz = coords_v[0][s]
                py = coords_v[1][s]
                px = coords_v[2][s]

                def split(f, n):
                    f = jnp.clip(f * float(n - 1), 0.0, float(n - 1))
                    i0 = f.astype(jnp.int32)
                    w1 = f - i0.astype(jnp.float32)
                    i1 = jnp.minimum(i0 + 1, n - 1)
                    return i0, i1, 1.0 - w1, w1

                z0, z1, wz0, wz1 = split(pz, D)
                y0, y1, wy0, wy1 = split(py, H)
                x0, x1, wx0, wx1 = split(px, W)
                zy = [(z0 * H + y0, wz0 * wy0), (z0 * H + y1, wz0 * wy1),
                      (z1 * H + y0, wz1 * wy0), (z1 * H + y1, wz1 * wy1)]
                sl = pl.ds(0, L)
                ci = 0
                for lzy, wzy in zy:
                    for lx, wx in ((x0, wx0), (x1, wx1)):
                        idx_v[par, ci, sl] = lzy * W + lx + (b * DHW)
                        w_v[par, ci, sl] = wzy * wx
                        ci += 1

            def issue(par):
                for ci in range(8):
                    pltpu.async_copy(table_hbm.at[idx_v.at[par, ci]],
                                     rows_v.at[par, ci], sems[par])

            def drain(par):
                for ci in range(8):
                    pltpu.make_async_copy(table_hbm.at[idx_v.at[par, ci]],
                                          rows_v.at[par, ci],
                                          sems[par]).wait()

            def out_slice(g, b=b):
                return out_hbm.at[pl.ds(b * n_pad + tile_base + g * CHUNK,
                                        CHUNK)]

            def blend_store(g, par, opar, b=b):
                @pl.when(g >= 2)
                def _():
                    pltpu.make_async_copy(out_v.at[opar], out_slice(g - 2),
                                          osems[opar]).wait()

                def blend(p, carry2):
                    ws = [w_v[par, ci, pl.ds(p, L)][0] for ci in range(8)]
                    for v in range(C // L):
                        t = [rows_v[par, ci, p, pl.ds(v * L, L)]
                             * ws[ci] for ci in range(8)]
                        out_v[opar, p, pl.ds(v * L, L)] = (
                            (t[0] + t[1]) + (t[2] + t[3])) + (
                            (t[4] + t[5]) + (t[6] + t[7]))
                    return carry2

                lax.fori_loop(0, CHUNK, blend, 0, unroll=2)
                pltpu.async_copy(out_v.at[opar], out_slice(g), osems[opar])

            fill_idx(0, 0)
            issue(0)

            def chunk_body(g0, carry):
                for par in range(2):
                    g = g0 * 2 + par

                    @pl.when(g + 1 < n_chunks)
                    def _():
                        fill_idx(g + 1, 1 - par)
                        issue(1 - par)

                    drain(par)
                    blend_store(g, par, par)
                return carry

            lax.fori_loop(0, n_chunks // 2, chunk_body, 0, unroll=False)

            pltpu.make_async_copy(out_v.at[0], out_slice(n_chunks - 2),
                                  osems[0]).wait()
            pltpu.make_async_copy(out_v.at[1], out_slice(n_chunks - 1),
                                  osems[1]).wait()

    return k(table, pc)


def kernel(voxel_features, voxel_coords, point_coords):
    del voxel_coords
    B, C, D, H, W = voxel_features.shape
    N = point_coords.shape[1]
    DHW = D * H * W
    per_tile = -(-N // NW)
    per_tile = -(-per_tile // (2 * CHUNK)) * (2 * CHUNK)
    n_pad = per_tile * NW

    table = voxel_features.transpose(0, 2, 3, 4, 1).reshape(B * DHW, C)
    pc = point_coords.transpose(0, 2, 1)
    pc = jnp.pad(pc, ((0, 0), (0, 0), (0, n_pad - N))).reshape(B * 3 * n_pad)
    out = _sc_interp(table, pc, B=B, D=D, H=H, W=W, C=C, n_pad=n_pad)
    return out.reshape(B, n_pad, C)[:, :N, :]
